# baseline (speedup 1.0000x reference)
import functools

import jax
import jax.numpy as jnp
from jax import lax
from jax.experimental import pallas as pl
from jax.experimental.pallas import tpu as pltpu
from jax.experimental.pallas import tpu_sc as plsc

BATCH = 16384
NUM_FACTORS = 32
WORDS = NUM_FACTORS // 2
PACK = 128 // WORDS
NUM_WORKERS = 32
PER_WORKER = BATCH // NUM_WORKERS
N_CHUNKS = 4
CHUNK = PER_WORKER // N_CHUNKS
GROUPS_PER_CHUNK = CHUNK // 16
HIMASK = -65536


def _sc_body(uid_hbm, iid_hbm, exp_hbm, uemb_hbm, iemb_hbm, gamma_hbm,
             bias_hbm, out_hbm,
             uid_v, iid_v, uidx_v, iidx_v, exp_v, gam_v, bias_v, out_v,
             urows0_v, urows1_v, irows0_v, irows1_v,
             gsem, usem0, usem1, isem0, isem1):
    n_cores = 2
    wid = lax.axis_index("s") * n_cores + lax.axis_index("c")
    base = wid * PER_WORKER

    pltpu.sync_copy(uid_hbm.at[pl.ds(N_CHUNKS * wid, N_CHUNKS)], uid_v)
    pltpu.sync_copy(iid_hbm.at[pl.ds(N_CHUNKS * wid, N_CHUNKS)], iid_v)
    pltpu.sync_copy(exp_hbm.at[pl.ds(base, PER_WORKER)], exp_v)
    pltpu.sync_copy(bias_hbm, bias_v)

    gamma_copies = [
        pltpu.async_copy(gamma_hbm.at[uid_v.at[j]],
                         gam_v.at[pl.ds(j * CHUNK, CHUNK)], gsem)
        for j in range(N_CHUNKS)
    ]

    for j in range(N_CHUNKS):
        for k in range(CHUNK // 16):
            sl = pl.ds(k * 16, 16)
            uidx_v[j, sl] = jax.lax.shift_right_logical(uid_v[j, sl], 3)
            iidx_v[j, sl] = jax.lax.shift_right_logical(iid_v[j, sl], 3)

    ubufs = [urows0_v, urows1_v]
    ibufs = [irows0_v, irows1_v]
    usems = [usem0, usem1]
    isems = [isem0, isem1]

    def fire(j):
        b = j % 2
        return (pltpu.async_copy(uemb_hbm.at[uidx_v.at[j]], ubufs[b], usems[b]),
                pltpu.async_copy(iemb_hbm.at[iidx_v.at[j]], ibufs[b], isems[b]))

    lane = lax.iota(jnp.int32, 16)
    bias_vec = bias_v[...]

    inflight = fire(0)
    gamma_copies[0].wait()
    gamma_copies[1].wait()
    gamma_copies[2].wait()
    gamma_copies[3].wait()

    for j in range(N_CHUNKS):
        uc, ic = inflight
        uc.wait()
        ic.wait()
        if j + 1 < N_CHUNKS:
            inflight = fire(j + 1)
        ub, ibuf = ubufs[j % 2], ibufs[j % 2]

        def group(g, _):
            row = g * 16 + lane
            off = j * CHUNK + g * 16
            uid16 = uid_v[j, pl.ds(g * 16, 16)]
            iid16 = iid_v[j, pl.ds(g * 16, 16)]
            ucol = (uid16 & (PACK - 1)) * WORDS
            icol = (iid16 & (PACK - 1)) * WORDS
            acc = gam_v[pl.ds(off, 16)] * exp_v[pl.ds(off, 16)] + bias_vec
            for w in range(WORDS):
                uw = plsc.load_gather(ub, [row, ucol + w])
                iw = plsc.load_gather(ibuf, [row, icol + w])
                ulo = plsc.bitcast(jax.lax.shift_left(uw, 16), jnp.float32)
                ilo = plsc.bitcast(jax.lax.shift_left(iw, 16), jnp.float32)
                uhi = plsc.bitcast(uw & HIMASK, jnp.float32)
                ihi = plsc.bitcast(iw & HIMASK, jnp.float32)
                acc = acc + ulo * ilo + uhi * ihi
            out_v[pl.ds(off, 16)] = acc
            return _

        lax.fori_loop(0, GROUPS_PER_CHUNK, group, None)

    pltpu.sync_copy(out_v, out_hbm.at[pl.ds(base, PER_WORKER)])


@jax.jit
def kernel(user_ids, item_ids, exposures_hat, user_embeddings,
           item_embeddings, gamma, bias):
    mesh = plsc.VectorSubcoreMesh(core_axis_name="c", subcore_axis_name="s")
    uid2 = user_ids.reshape(BATCH // CHUNK, CHUNK)
    iid2 = item_ids.reshape(BATCH // CHUNK, CHUNK)

    def pack(t):
        rows = t.shape[0] // PACK
        return jax.lax.bitcast_convert_type(
            t.astype(jnp.bfloat16).reshape(rows, 128, 2), jnp.int32)

    um = pack(user_embeddings)
    im = pack(item_embeddings)
    bias16 = jnp.broadcast_to(bias, (16,))
    run = functools.partial(
        pl.kernel,
        mesh=mesh,
        compiler_params=pltpu.CompilerParams(
            needs_layout_passes=False, use_tc_tiling_on_sc=True),
        out_type=jax.ShapeDtypeStruct((BATCH,), jnp.float32),
        scratch_types=[
            pltpu.VMEM((N_CHUNKS, CHUNK), jnp.int32),
            pltpu.VMEM((N_CHUNKS, CHUNK), jnp.int32),
            pltpu.VMEM((N_CHUNKS, CHUNK), jnp.int32),
            pltpu.VMEM((N_CHUNKS, CHUNK), jnp.int32),
            pltpu.VMEM((PER_WORKER,), jnp.float32),
            pltpu.VMEM((PER_WORKER,), jnp.float32),
            pltpu.VMEM((16,), jnp.float32),
            pltpu.VMEM((PER_WORKER,), jnp.float32),
            pltpu.VMEM((CHUNK, 128), jnp.int32),
            pltpu.VMEM((CHUNK, 128), jnp.int32),
            pltpu.VMEM((CHUNK, 128), jnp.int32),
            pltpu.VMEM((CHUNK, 128), jnp.int32),
            pltpu.SemaphoreType.DMA,
            pltpu.SemaphoreType.DMA,
            pltpu.SemaphoreType.DMA,
            pltpu.SemaphoreType.DMA,
            pltpu.SemaphoreType.DMA,
        ],
    )(_sc_body)
    return run(uid2, iid2, exposures_hat, um, im, gamma, bias16)


# speedup vs baseline: 13.0799x; 13.0799x over previous
import functools

import jax
import jax.numpy as jnp
from jax import lax
from jax.experimental import pallas as pl
from jax.experimental.pallas import tpu as pltpu
from jax.experimental.pallas import tpu_sc as plsc

BATCH = 16384
NUM_FACTORS = 32
WORDS = NUM_FACTORS // 2
PACK = 128 // WORDS
NUM_WORKERS = 32
PER_WORKER = BATCH // NUM_WORKERS
N_CHUNKS = 4
CHUNK = PER_WORKER // N_CHUNKS
GROUPS_PER_CHUNK = CHUNK // 16
HIMASK = -65536


def _sc_body(uid_hbm, iid_hbm, exp_hbm, uemb_hbm, iemb_hbm, gamma_hbm,
             bias_hbm, out_hbm,
             uid_v, iid_v, uidx_v, iidx_v, exp_v, gam_v, bias_v, out_v,
             urows0_v, urows1_v, irows0_v, irows1_v,
             gsem, usem0, usem1, isem0, isem1):
    n_cores = 2
    wid = lax.axis_index("s") * n_cores + lax.axis_index("c")
    base = wid * PER_WORKER

    pltpu.sync_copy(uid_hbm.at[pl.ds(N_CHUNKS * wid, N_CHUNKS)], uid_v)
    pltpu.sync_copy(iid_hbm.at[pl.ds(N_CHUNKS * wid, N_CHUNKS)], iid_v)
    pltpu.sync_copy(exp_hbm.at[pl.ds(base, PER_WORKER)], exp_v)
    pltpu.sync_copy(bias_hbm, bias_v)

    gamma_copies = [
        pltpu.async_copy(gamma_hbm.at[uid_v.at[j]],
                         gam_v.at[pl.ds(j * CHUNK, CHUNK)], gsem)
        for j in range(N_CHUNKS)
    ]

    for j in range(N_CHUNKS):
        for k in range(CHUNK // 16):
            sl = pl.ds(k * 16, 16)
            uidx_v[j, sl] = jax.lax.shift_right_logical(uid_v[j, sl], 3)
            iidx_v[j, sl] = jax.lax.shift_right_logical(iid_v[j, sl], 3)

    ubufs = [urows0_v, urows1_v]
    ibufs = [irows0_v, irows1_v]
    usems = [usem0, usem1]
    isems = [isem0, isem1]

    def fire(j):
        b = j % 2
        return (pltpu.async_copy(uemb_hbm.at[uidx_v.at[j]], ubufs[b], usems[b]),
                pltpu.async_copy(iemb_hbm.at[iidx_v.at[j]], ibufs[b], isems[b]))

    lane = lax.iota(jnp.int32, 16)
    bias_vec = bias_v[...]

    inflight = fire(0)
    gamma_copies[0].wait()
    gamma_copies[1].wait()
    gamma_copies[2].wait()
    gamma_copies[3].wait()

    for j in range(N_CHUNKS):
        uc, ic = inflight
        uc.wait()
        ic.wait()
        if j + 1 < N_CHUNKS:
            inflight = fire(j + 1)
        ub, ibuf = ubufs[j % 2], ibufs[j % 2]

        def group(g, _):
            row = g * 16 + lane
            off = j * CHUNK + g * 16
            uid16 = uid_v[j, pl.ds(g * 16, 16)]
            iid16 = iid_v[j, pl.ds(g * 16, 16)]
            ucol = (uid16 & (PACK - 1)) * WORDS
            icol = (iid16 & (PACK - 1)) * WORDS
            acc = gam_v[pl.ds(off, 16)] * exp_v[pl.ds(off, 16)] + bias_vec
            for w in range(WORDS):
                uw = plsc.load_gather(ub, [row, ucol + w])
                iw = plsc.load_gather(ibuf, [row, icol + w])
                ulo = plsc.bitcast(jax.lax.shift_left(uw, 16), jnp.float32)
                ilo = plsc.bitcast(jax.lax.shift_left(iw, 16), jnp.float32)
                uhi = plsc.bitcast(uw & HIMASK, jnp.float32)
                ihi = plsc.bitcast(iw & HIMASK, jnp.float32)
                acc = acc + ulo * ilo + uhi * ihi
            out_v[pl.ds(off, 16)] = acc
            return _

        lax.fori_loop(0, GROUPS_PER_CHUNK, group, None)

    pltpu.sync_copy(out_v, out_hbm.at[pl.ds(base, PER_WORKER)])


@jax.jit
def kernel(user_ids, item_ids, exposures_hat, user_embeddings,
           item_embeddings, gamma, bias):
    mesh = plsc.VectorSubcoreMesh(core_axis_name="c", subcore_axis_name="s")
    uid2 = user_ids.reshape(BATCH // CHUNK, CHUNK)
    iid2 = item_ids.reshape(BATCH // CHUNK, CHUNK)

    def pack(t):
        v = jax.lax.bitcast_convert_type(t, jnp.uint32)
        r = v + jnp.uint32(0x7FFF) + ((v >> 16) & jnp.uint32(1))
        w = (r[:, :WORDS] >> 16) | (r[:, WORDS:] & jnp.uint32(0xFFFF0000))
        return jax.lax.bitcast_convert_type(w, jnp.int32).reshape(
            t.shape[0] // PACK, 128)

    um = pack(user_embeddings)
    im = pack(item_embeddings)
    bias16 = jnp.broadcast_to(bias, (16,))
    run = functools.partial(
        pl.kernel,
        mesh=mesh,
        compiler_params=pltpu.CompilerParams(
            needs_layout_passes=False, use_tc_tiling_on_sc=True),
        out_type=jax.ShapeDtypeStruct((BATCH,), jnp.float32),
        scratch_types=[
            pltpu.VMEM((N_CHUNKS, CHUNK), jnp.int32),
            pltpu.VMEM((N_CHUNKS, CHUNK), jnp.int32),
            pltpu.VMEM((N_CHUNKS, CHUNK), jnp.int32),
            pltpu.VMEM((N_CHUNKS, CHUNK), jnp.int32),
            pltpu.VMEM((PER_WORKER,), jnp.float32),
            pltpu.VMEM((PER_WORKER,), jnp.float32),
            pltpu.VMEM((16,), jnp.float32),
            pltpu.VMEM((PER_WORKER,), jnp.float32),
            pltpu.VMEM((CHUNK, 128), jnp.int32),
            pltpu.VMEM((CHUNK, 128), jnp.int32),
            pltpu.VMEM((CHUNK, 128), jnp.int32),
            pltpu.VMEM((CHUNK, 128), jnp.int32),
            pltpu.SemaphoreType.DMA,
            pltpu.SemaphoreType.DMA,
            pltpu.SemaphoreType.DMA,
            pltpu.SemaphoreType.DMA,
            pltpu.SemaphoreType.DMA,
        ],
    )(_sc_body)
    return run(uid2, iid2, exposures_hat, um, im, gamma, bias16)


# speedup vs baseline: 15.2311x; 1.1645x over previous
import functools

import jax
import jax.numpy as jnp
from jax import lax
from jax.experimental import pallas as pl
from jax.experimental.pallas import tpu as pltpu
from jax.experimental.pallas import tpu_sc as plsc

BATCH = 16384
NUM_FACTORS = 32
NUM_WORKERS = 32
PER_WORKER = BATCH // NUM_WORKERS
N_CHUNKS = 4
CHUNK = PER_WORKER // N_CHUNKS
GROUPS = PER_WORKER // 16


def _sc_body(uid_hbm, iid_hbm, exp_hbm, uemb_hbm, iemb_hbm, gamma_hbm,
             bias_hbm, out_hbm,
             uid_v, iid_v, exp_v, urows_v, irows_v, gam_v, bias_v, out_v,
             sem, sem2):
    n_cores = 2
    wid = lax.axis_index("s") * n_cores + lax.axis_index("c")
    base = wid * PER_WORKER

    pltpu.sync_copy(uid_hbm.at[wid], uid_v)
    pltpu.sync_copy(iid_hbm.at[wid], iid_v)
    pltpu.sync_copy(exp_hbm.at[pl.ds(base, PER_WORKER)], exp_v)
    pltpu.sync_copy(bias_hbm, bias_v)

    copies = []
    for j in range(N_CHUNKS):
        copies.append(pltpu.async_copy(
            uemb_hbm.at[uid_v.at[j]],
            urows_v.at[pl.ds(j * CHUNK, CHUNK)], sem))
        copies.append(pltpu.async_copy(
            iemb_hbm.at[iid_v.at[j]],
            irows_v.at[pl.ds(j * CHUNK, CHUNK)], sem))
        copies.append(pltpu.async_copy(
            gamma_hbm.at[uid_v.at[j]],
            gam_v.at[pl.ds(j * CHUNK, CHUNK)], sem2))
    for c in copies:
        c.wait()

    lane = lax.iota(jnp.int32, 16)
    bias_vec = bias_v[...]

    def group(g, _):
        row = g * 16 + lane
        acc = gam_v[pl.ds(g * 16, 16)] * exp_v[pl.ds(g * 16, 16)] + bias_vec
        for d in range(NUM_FACTORS):
            col = jnp.full((16,), d, jnp.int32)
            u = plsc.load_gather(urows_v, [row, col])
            v = plsc.load_gather(irows_v, [row, col])
            acc = acc + u * v
        out_v[pl.ds(g * 16, 16)] = acc
        return _

    lax.fori_loop(0, GROUPS, group, None)

    pltpu.sync_copy(out_v, out_hbm.at[pl.ds(base, PER_WORKER)])


@jax.jit
def kernel(user_ids, item_ids, exposures_hat, user_embeddings,
           item_embeddings, gamma, bias):
    mesh = plsc.VectorSubcoreMesh(core_axis_name="c", subcore_axis_name="s")
    uid3 = user_ids.reshape(NUM_WORKERS, N_CHUNKS, CHUNK)
    iid3 = item_ids.reshape(NUM_WORKERS, N_CHUNKS, CHUNK)
    bias16 = jnp.broadcast_to(bias, (16,))
    run = functools.partial(
        pl.kernel,
        mesh=mesh,
        compiler_params=pltpu.CompilerParams(
            needs_layout_passes=False, use_tc_tiling_on_sc=False),
        out_type=jax.ShapeDtypeStruct((BATCH,), jnp.float32),
        scratch_types=[
            pltpu.VMEM((N_CHUNKS, CHUNK), jnp.int32),
            pltpu.VMEM((N_CHUNKS, CHUNK), jnp.int32),
            pltpu.VMEM((PER_WORKER,), jnp.float32),
            pltpu.VMEM((PER_WORKER, NUM_FACTORS), jnp.float32),
            pltpu.VMEM((PER_WORKER, NUM_FACTORS), jnp.float32),
            pltpu.VMEM((PER_WORKER,), jnp.float32),
            pltpu.VMEM((16,), jnp.float32),
            pltpu.VMEM((PER_WORKER,), jnp.float32),
            pltpu.SemaphoreType.DMA,
            pltpu.SemaphoreType.DMA,
        ],
    )(_sc_body)
    return run(uid3, iid3, exposures_hat, user_embeddings, item_embeddings,
               gamma, bias16)
